# baseline (speedup 1.0000x reference)
import jax
import jax.numpy as jnp
from jax.experimental import pallas as pl
from jax.experimental.pallas import tpu as pltpu

_C = 4608
_NCHUNK = 8


def _body(m_ref, t_ref, x_hbm, o_hbm, xb0, xb1, ob0, ob1,
          si0, si1, so0, so1):
    xbs = (xb0, xb1)
    obs = (ob0, ob1)
    sis = (si0, si1)
    sos = (so0, so1)
    tok = t_ref[0, :]

    def start_in(i):
        b = i % 2
        pltpu.make_async_copy(x_hbm.at[pl.ds(i * _C, _C), :], xbs[b], sis[b]).start()

    def wait_in(i):
        b = i % 2
        pltpu.make_async_copy(x_hbm.at[pl.ds(i * _C, _C), :], xbs[b], sis[b]).wait()

    def start_out(i):
        b = i % 2
        pltpu.make_async_copy(obs[b], o_hbm.at[pl.ds(i * _C, _C), :], sos[b]).start()

    def wait_out(i):
        b = i % 2
        pltpu.make_async_copy(obs[b], o_hbm.at[pl.ds(i * _C, _C), :], sos[b]).wait()

    start_in(0)
    start_in(1)
    for i in range(_NCHUNK):
        b = i % 2
        wait_in(i)
        if i >= 2:
            wait_out(i - 2)
        m = m_ref[i, :].astype(jnp.int32)
        obs[b][:, :] = jnp.where(m[:, None] != 0, tok[None, :], xbs[b][:, :])
        start_out(i)
        if i + 2 < _NCHUNK:
            start_in(i + 2)
    wait_out(_NCHUNK - 2)
    wait_out(_NCHUNK - 1)


def kernel(x, patch_mask, mask_token):
    N, L, H, W, D = x.shape
    rows = N * L * H * W
    xf = x.reshape(rows, D)
    mf = patch_mask.reshape(_NCHUNK, rows // _NCHUNK)

    out = pl.pallas_call(
        _body,
        in_specs=[
            pl.BlockSpec(memory_space=pltpu.MemorySpace.VMEM),
            pl.BlockSpec(memory_space=pltpu.MemorySpace.VMEM),
            pl.BlockSpec(memory_space=pltpu.MemorySpace.HBM),
        ],
        out_specs=pl.BlockSpec(memory_space=pltpu.MemorySpace.HBM),
        out_shape=jax.ShapeDtypeStruct((rows, D), x.dtype),
        scratch_shapes=[
            pltpu.VMEM((_C, D), jnp.float32),
            pltpu.VMEM((_C, D), jnp.float32),
            pltpu.VMEM((_C, D), jnp.float32),
            pltpu.VMEM((_C, D), jnp.float32),
            pltpu.SemaphoreType.DMA,
            pltpu.SemaphoreType.DMA,
            pltpu.SemaphoreType.DMA,
            pltpu.SemaphoreType.DMA,
        ],
    )(mf, mask_token, xf)

    return (out.reshape(x.shape), patch_mask)


# speedup vs baseline: 1.0139x; 1.0139x over previous
import jax
import jax.numpy as jnp
from jax.experimental import pallas as pl
from jax.experimental.pallas import tpu as pltpu

_ROWS_PER_BLOCK = 4608


def _select_body(m_ref, t_ref, x_ref, o_ref):
    i = pl.program_id(0)
    m = m_ref[i, :].astype(jnp.int32)
    tok = t_ref[0, :]
    o_ref[:, :] = jnp.where(m[:, None] != 0, tok[None, :], x_ref[:, :])


def kernel(x, patch_mask, mask_token):
    N, L, H, W, D = x.shape
    rows = N * L * H * W
    xf = x.reshape(rows, D)
    nblk = rows // _ROWS_PER_BLOCK
    mf = patch_mask.reshape(nblk, _ROWS_PER_BLOCK)

    out = pl.pallas_call(
        _select_body,
        grid=(nblk,),
        in_specs=[
            pl.BlockSpec((nblk, _ROWS_PER_BLOCK), lambda i: (0, 0)),
            pl.BlockSpec((1, D), lambda i: (0, 0)),
            pl.BlockSpec((_ROWS_PER_BLOCK, D), lambda i: (i, 0)),
        ],
        out_specs=pl.BlockSpec((_ROWS_PER_BLOCK, D), lambda i: (i, 0)),
        out_shape=jax.ShapeDtypeStruct((rows, D), x.dtype),
        compiler_params=pltpu.CompilerParams(
            dimension_semantics=("arbitrary",),
        ),
    )(mf, mask_token, xf)

    return (out.reshape(x.shape), patch_mask)
